# Initial kernel scaffold; baseline (speedup 1.0000x reference)
import functools

import jax
import jax.numpy as jnp
from jax import lax
from jax.experimental import pallas as pl
from jax.experimental.pallas import tpu as pltpu
from jax.experimental.pallas import tpu_sc as plsc

N_NODES = 10000
N_EDGES = 320000
D_FEAT = 128
D_OUT = 64

NC = 2
NS = 16
L = 16
NW = NC * NS
E_PER_W = N_EDGES // NW
CHUNK = 80
N_CHUNKS = E_PER_W // CHUNK
RPS = N_NODES // NS
ZROWS = 125

_mesh = plsc.VectorSubcoreMesh(core_axis_name="c", subcore_axis_name="s")


@functools.partial(
    pl.kernel,
    out_type=(
        jax.ShapeDtypeStruct((NC, N_NODES, D_OUT), jnp.float32),
        jax.ShapeDtypeStruct((NC, N_NODES, L), jnp.float32),
    ),
    mesh=_mesh,
    scratch_types=[
        pltpu.VMEM((CHUNK,), jnp.int32),
        pltpu.VMEM((CHUNK,), jnp.int32),
        pltpu.VMEM((CHUNK, D_OUT), jnp.float32),
        pltpu.VMEM((CHUNK, L), jnp.float32),
        pltpu.VMEM((ZROWS, D_OUT), jnp.float32),
        pltpu.VMEM((ZROWS, L), jnp.float32),
        pltpu.VMEM_SHARED((N_NODES, D_OUT), jnp.float32),
        pltpu.VMEM_SHARED((N_NODES, L), jnp.float32),
    ],
)
def _sc_segment_sum(y_hbm, src_hbm, dst_hbm, sum_hbm, cnt_hbm,
                    src_v, dst_v, rows_v, ones_v, z64_v, z16_v,
                    acc_sh, cnt_sh):
    cid = lax.axis_index("c")
    sid = lax.axis_index("s")
    wid = sid * NC + cid

    @pl.loop(0, CHUNK)
    def _(i):
        ones_v.at[i][...] = jnp.ones((L,), jnp.float32)

    @pl.loop(0, ZROWS)
    def _(i):
        z16_v.at[i][...] = jnp.zeros((L,), jnp.float32)

        @pl.loop(0, D_OUT, step=L)
        def _(j):
            z64_v.at[i, pl.ds(j, L)][...] = jnp.zeros((L,), jnp.float32)

    @pl.loop(0, RPS, step=ZROWS)
    def _(k):
        pltpu.sync_copy(z64_v, acc_sh.at[pl.ds(sid * RPS + k, ZROWS)])
        pltpu.sync_copy(z16_v, cnt_sh.at[pl.ds(sid * RPS + k, ZROWS)])

    plsc.subcore_barrier()

    base = wid * E_PER_W

    @pl.loop(0, N_CHUNKS)
    def _(t):
        off = base + t * CHUNK
        pltpu.sync_copy(src_hbm.at[pl.ds(off, CHUNK)], src_v)
        pltpu.sync_copy(dst_hbm.at[pl.ds(off, CHUNK)], dst_v)
        pltpu.sync_copy(y_hbm.at[src_v], rows_v)
        pltpu.sync_copy(rows_v, acc_sh.at[dst_v], add=True)
        pltpu.sync_copy(ones_v, cnt_sh.at[dst_v], add=True)

    plsc.subcore_barrier()

    pltpu.sync_copy(acc_sh.at[pl.ds(sid * RPS, RPS)],
                    sum_hbm.at[cid, pl.ds(sid * RPS, RPS)])
    pltpu.sync_copy(cnt_sh.at[pl.ds(sid * RPS, RPS)],
                    cnt_hbm.at[cid, pl.ds(sid * RPS, RPS)])


_BLK = 1000


def _pre_body(x_ref, wl_ref, wr_ref, b_ref, y_ref, r_ref):
    xb = x_ref[...]
    dn = (((1,), (1,)), ((), ()))
    y_ref[...] = lax.dot_general(xb, wl_ref[...], dn,
                                 precision=lax.Precision.HIGHEST,
                                 preferred_element_type=jnp.float32)
    r_ref[...] = lax.dot_general(xb, wr_ref[...], dn,
                                 precision=lax.Precision.HIGHEST,
                                 preferred_element_type=jnp.float32) + b_ref[...]


def _post_body(s_ref, c_ref, r_ref, o_ref):
    s = s_ref[0] + s_ref[1]
    c = c_ref[...]
    deg = jnp.maximum(c[0, :, 0:1] + c[1, :, 0:1], 1.0)
    o = s / deg + r_ref[...]
    m = jnp.max(o, axis=1, keepdims=True)
    e = o - m
    lse = jnp.log(jnp.sum(jnp.exp(e), axis=1, keepdims=True))
    o_ref[...] = e - lse


def kernel(x, index, W_l, W_r, b):
    idx = index.astype(jnp.int32)
    src = idx[0]
    dst = idx[1]
    b2 = b.reshape(1, D_OUT).astype(jnp.float32)

    grid = N_NODES // _BLK
    y, r = pl.pallas_call(
        _pre_body,
        grid=(grid,),
        in_specs=[
            pl.BlockSpec((_BLK, D_FEAT), lambda i: (i, 0)),
            pl.BlockSpec((D_OUT, D_FEAT), lambda i: (0, 0)),
            pl.BlockSpec((D_OUT, D_FEAT), lambda i: (0, 0)),
            pl.BlockSpec((1, D_OUT), lambda i: (0, 0)),
        ],
        out_specs=[
            pl.BlockSpec((_BLK, D_OUT), lambda i: (i, 0)),
            pl.BlockSpec((_BLK, D_OUT), lambda i: (i, 0)),
        ],
        out_shape=[
            jax.ShapeDtypeStruct((N_NODES, D_OUT), jnp.float32),
            jax.ShapeDtypeStruct((N_NODES, D_OUT), jnp.float32),
        ],
    )(x, W_l, W_r, b2)

    sums, cnts = _sc_segment_sum(y, src, dst)

    out = pl.pallas_call(
        _post_body,
        grid=(grid,),
        in_specs=[
            pl.BlockSpec((NC, _BLK, D_OUT), lambda i: (0, i, 0)),
            pl.BlockSpec((NC, _BLK, L), lambda i: (0, i, 0)),
            pl.BlockSpec((_BLK, D_OUT), lambda i: (i, 0)),
        ],
        out_specs=pl.BlockSpec((_BLK, D_OUT), lambda i: (i, 0)),
        out_shape=jax.ShapeDtypeStruct((N_NODES, D_OUT), jnp.float32),
    )(sums, cnts, r)

    return out


# speedup vs baseline: 6.5451x; 6.5451x over previous
import functools

import jax
import jax.numpy as jnp
from jax import lax
from jax.experimental import pallas as pl
from jax.experimental.pallas import tpu as pltpu
from jax.experimental.pallas import tpu_sc as plsc

N_NODES = 10000
N_EDGES = 320000
D_FEAT = 128
D_OUT = 64

NC = 2
NS = 16
L = 16
NW = NC * NS
E_PER_W = N_EDGES // NW
CHUNK = 80
N_CHUNKS = E_PER_W // CHUNK
N_PAD = 10240
RPS = N_PAD // NS
ZROWS = 128

_mesh = plsc.VectorSubcoreMesh(core_axis_name="c", subcore_axis_name="s")


@functools.partial(
    pl.kernel,
    out_type=(
        jax.ShapeDtypeStruct((NC, N_PAD, D_OUT), jnp.float32),
        jax.ShapeDtypeStruct((NC, N_PAD, L), jnp.float32),
    ),
    mesh=_mesh,
    compiler_params=pltpu.CompilerParams(use_tc_tiling_on_sc=False),
    scratch_types=[
        pltpu.VMEM((CHUNK,), jnp.int32),
        pltpu.VMEM((CHUNK,), jnp.int32),
        pltpu.VMEM((CHUNK, D_OUT), jnp.float32),
        pltpu.VMEM((CHUNK, L), jnp.float32),
        pltpu.VMEM((ZROWS, D_OUT), jnp.float32),
        pltpu.VMEM((ZROWS, L), jnp.float32),
        pltpu.VMEM_SHARED((N_PAD, D_OUT), jnp.float32),
        pltpu.VMEM_SHARED((N_PAD, L), jnp.float32),
    ],
)
def _sc_segment_sum(y_hbm, src_hbm, dst_hbm, sum_hbm, cnt_hbm,
                    src_v, dst_v, rows_v, ones_v, z64_v, z16_v,
                    acc_sh, cnt_sh):
    cid = lax.axis_index("c")
    sid = lax.axis_index("s")
    wid = sid * NC + cid

    @pl.loop(0, CHUNK)
    def _(i):
        ones_v.at[i][...] = jnp.ones((L,), jnp.float32)

    @pl.loop(0, ZROWS)
    def _(i):
        z16_v.at[i][...] = jnp.zeros((L,), jnp.float32)

        @pl.loop(0, D_OUT, step=L)
        def _(j):
            z64_v.at[i, pl.ds(j, L)][...] = jnp.zeros((L,), jnp.float32)

    @pl.loop(0, RPS, step=ZROWS)
    def _(k):
        pltpu.sync_copy(z64_v, acc_sh.at[pl.ds(sid * RPS + k, ZROWS)])
        pltpu.sync_copy(z16_v, cnt_sh.at[pl.ds(sid * RPS + k, ZROWS)])

    plsc.subcore_barrier()

    base = wid * E_PER_W

    @pl.loop(0, N_CHUNKS)
    def _(t):
        off = base + t * CHUNK
        pltpu.sync_copy(src_hbm.at[pl.ds(off, CHUNK)], src_v)
        pltpu.sync_copy(dst_hbm.at[pl.ds(off, CHUNK)], dst_v)
        pltpu.sync_copy(y_hbm.at[src_v], rows_v)
        pltpu.sync_copy(rows_v, acc_sh.at[dst_v], add=True)
        pltpu.sync_copy(ones_v, cnt_sh.at[dst_v], add=True)

    plsc.subcore_barrier()

    pltpu.sync_copy(acc_sh.at[pl.ds(sid * RPS, RPS)],
                    sum_hbm.at[cid, pl.ds(sid * RPS, RPS)])
    pltpu.sync_copy(cnt_sh.at[pl.ds(sid * RPS, RPS)],
                    cnt_hbm.at[cid, pl.ds(sid * RPS, RPS)])


_BLK = 1000


def _pre_body(x_ref, wl_ref, wr_ref, b_ref, y_ref, r_ref):
    xb = x_ref[...]
    dn = (((1,), (1,)), ((), ()))
    y_ref[...] = lax.dot_general(xb, wl_ref[...], dn,
                                 precision=lax.Precision.HIGHEST,
                                 preferred_element_type=jnp.float32)
    r_ref[...] = lax.dot_general(xb, wr_ref[...], dn,
                                 precision=lax.Precision.HIGHEST,
                                 preferred_element_type=jnp.float32) + b_ref[...]


def _post_body(s_ref, c_ref, r_ref, o_ref):
    s = s_ref[0] + s_ref[1]
    c = c_ref[...]
    deg = jnp.maximum(c[0, :, 0:1] + c[1, :, 0:1], 1.0)
    o = s / deg + r_ref[...]
    m = jnp.max(o, axis=1, keepdims=True)
    e = o - m
    lse = jnp.log(jnp.sum(jnp.exp(e), axis=1, keepdims=True))
    o_ref[...] = e - lse


def kernel(x, index, W_l, W_r, b):
    idx = index.astype(jnp.int32)
    src = idx[0]
    dst = idx[1]
    b2 = b.reshape(1, D_OUT).astype(jnp.float32)

    grid = N_NODES // _BLK
    y, r = pl.pallas_call(
        _pre_body,
        grid=(grid,),
        in_specs=[
            pl.BlockSpec((_BLK, D_FEAT), lambda i: (i, 0)),
            pl.BlockSpec((D_OUT, D_FEAT), lambda i: (0, 0)),
            pl.BlockSpec((D_OUT, D_FEAT), lambda i: (0, 0)),
            pl.BlockSpec((1, D_OUT), lambda i: (0, 0)),
        ],
        out_specs=[
            pl.BlockSpec((_BLK, D_OUT), lambda i: (i, 0)),
            pl.BlockSpec((_BLK, D_OUT), lambda i: (i, 0)),
        ],
        out_shape=[
            jax.ShapeDtypeStruct((N_NODES, D_OUT), jnp.float32),
            jax.ShapeDtypeStruct((N_NODES, D_OUT), jnp.float32),
        ],
    )(x, W_l, W_r, b2)

    sums, cnts = _sc_segment_sum(y, src, dst)

    out = pl.pallas_call(
        _post_body,
        grid=(grid,),
        in_specs=[
            pl.BlockSpec((NC, _BLK, D_OUT), lambda i: (0, i, 0)),
            pl.BlockSpec((NC, _BLK, L), lambda i: (0, i, 0)),
            pl.BlockSpec((_BLK, D_OUT), lambda i: (i, 0)),
        ],
        out_specs=pl.BlockSpec((_BLK, D_OUT), lambda i: (i, 0)),
        out_shape=jax.ShapeDtypeStruct((N_NODES, D_OUT), jnp.float32),
    )(sums, cnts, r)

    return out


# speedup vs baseline: 13.7124x; 2.0951x over previous
import functools

import jax
import jax.numpy as jnp
from jax import lax
from jax.experimental import pallas as pl
from jax.experimental.pallas import tpu as pltpu
from jax.experimental.pallas import tpu_sc as plsc

N_NODES = 10000
N_EDGES = 320000
D_FEAT = 128
D_OUT = 64
D_ACC = 80

NC = 2
NS = 16
L = 16
NW = NC * NS
E_PER_W = N_EDGES // NW
CHUNK = 80
N_CHUNKS = E_PER_W // CHUNK
NBUF = 5
N_GROUPS = N_CHUNKS // NBUF
N_PAD = 10240
RPS = N_PAD // NS
ZROWS = 128

_mesh = plsc.VectorSubcoreMesh(core_axis_name="c", subcore_axis_name="s")


@functools.partial(
    pl.kernel,
    out_type=jax.ShapeDtypeStruct((NC, N_PAD, D_ACC), jnp.float32),
    mesh=_mesh,
    compiler_params=pltpu.CompilerParams(use_tc_tiling_on_sc=False),
    scratch_types=[
        pltpu.VMEM((N_CHUNKS, CHUNK), jnp.int32),
        pltpu.VMEM((N_CHUNKS, CHUNK), jnp.int32),
        [pltpu.VMEM((CHUNK, D_ACC), jnp.float32) for _ in range(NBUF)],
        pltpu.VMEM((ZROWS, D_ACC), jnp.float32),
        pltpu.VMEM_SHARED((N_PAD, D_ACC), jnp.float32),
        [pltpu.SemaphoreType.DMA for _ in range(NBUF)],
        [pltpu.SemaphoreType.DMA for _ in range(NBUF)],
    ],
)
def _sc_segment_sum(y2_hbm, src_hbm, dst_hbm, sum_hbm,
                    srcv, dstv, rows, zbuf, acc_sh, gsem, ssem):
    cid = lax.axis_index("c")
    sid = lax.axis_index("s")
    wid = sid * NC + cid

    @pl.loop(0, ZROWS)
    def _(i):
        @pl.loop(0, D_ACC, step=L)
        def _(j):
            zbuf.at[i, pl.ds(j, L)][...] = jnp.zeros((L,), jnp.float32)

    @pl.loop(0, RPS, step=ZROWS)
    def _(k):
        pltpu.sync_copy(zbuf, acc_sh.at[pl.ds(sid * RPS + k, ZROWS)])

    pltpu.sync_copy(src_hbm.at[wid], srcv)
    pltpu.sync_copy(dst_hbm.at[wid], dstv)

    plsc.subcore_barrier()

    def wait_bytes(sem, buf):
        pltpu.make_async_copy(y2_hbm.at[pl.ds(0, CHUNK)], buf, sem).wait()

    for b in range(NBUF):
        pltpu.async_copy(y2_hbm.at[srcv.at[b]], rows[b], gsem[b])

    @pl.loop(0, N_GROUPS)
    def _(g):
        t0 = g * NBUF
        for b in range(NBUF):
            wait_bytes(gsem[b], rows[b])
            pltpu.async_copy(rows[b], acc_sh.at[dstv.at[t0 + b]], ssem[b],
                             add=True)

        @pl.when(g < N_GROUPS - 1)
        def _():
            for b in range(NBUF):
                wait_bytes(ssem[b], rows[b])
                pltpu.async_copy(y2_hbm.at[srcv.at[t0 + NBUF + b]], rows[b],
                                 gsem[b])

    for b in range(NBUF):
        wait_bytes(ssem[b], rows[b])

    plsc.subcore_barrier()

    pltpu.sync_copy(acc_sh.at[pl.ds(sid * RPS, RPS)],
                    sum_hbm.at[cid, pl.ds(sid * RPS, RPS)])


_BLK = 1000


def _pre_body(x_ref, wl_ref, wr_ref, b_ref, y2_ref, r_ref):
    xb = x_ref[...]
    dn = (((1,), (1,)), ((), ()))
    yl = lax.dot_general(xb, wl_ref[...], dn,
                         precision=lax.Precision.HIGHEST,
                         preferred_element_type=jnp.float32)
    y2_ref[...] = jnp.concatenate(
        [yl, jnp.ones((_BLK, D_ACC - D_OUT), jnp.float32)], axis=1)
    r_ref[...] = lax.dot_general(xb, wr_ref[...], dn,
                                 precision=lax.Precision.HIGHEST,
                                 preferred_element_type=jnp.float32) + b_ref[...]


def _post_body(s_ref, r_ref, o_ref):
    s = s_ref[0] + s_ref[1]
    deg = jnp.maximum(s[:, D_OUT:D_OUT + 1], 1.0)
    o = s[:, :D_OUT] / deg + r_ref[...]
    m = jnp.max(o, axis=1, keepdims=True)
    e = o - m
    lse = jnp.log(jnp.sum(jnp.exp(e), axis=1, keepdims=True))
    o_ref[...] = e - lse


def kernel(x, index, W_l, W_r, b):
    idx = index.astype(jnp.int32).reshape(2, NW, N_CHUNKS, CHUNK)
    src = idx[0]
    dst = idx[1]
    b2 = b.reshape(1, D_OUT).astype(jnp.float32)

    grid = N_NODES // _BLK
    y2, r = pl.pallas_call(
        _pre_body,
        grid=(grid,),
        in_specs=[
            pl.BlockSpec((_BLK, D_FEAT), lambda i: (i, 0)),
            pl.BlockSpec((D_OUT, D_FEAT), lambda i: (0, 0)),
            pl.BlockSpec((D_OUT, D_FEAT), lambda i: (0, 0)),
            pl.BlockSpec((1, D_OUT), lambda i: (0, 0)),
        ],
        out_specs=[
            pl.BlockSpec((_BLK, D_ACC), lambda i: (i, 0)),
            pl.BlockSpec((_BLK, D_OUT), lambda i: (i, 0)),
        ],
        out_shape=[
            jax.ShapeDtypeStruct((N_NODES, D_ACC), jnp.float32),
            jax.ShapeDtypeStruct((N_NODES, D_OUT), jnp.float32),
        ],
    )(x, W_l, W_r, b2)

    sums = _sc_segment_sum(y2, src, dst)

    out = pl.pallas_call(
        _post_body,
        grid=(grid,),
        in_specs=[
            pl.BlockSpec((NC, _BLK, D_ACC), lambda i: (0, i, 0)),
            pl.BlockSpec((_BLK, D_OUT), lambda i: (i, 0)),
        ],
        out_specs=pl.BlockSpec((_BLK, D_OUT), lambda i: (i, 0)),
        out_shape=jax.ShapeDtypeStruct((N_NODES, D_OUT), jnp.float32),
    )(sums, r)

    return out


# speedup vs baseline: 21.2317x; 1.5484x over previous
import functools

import jax
import jax.numpy as jnp
from jax import lax
from jax.experimental import pallas as pl
from jax.experimental.pallas import tpu as pltpu
from jax.experimental.pallas import tpu_sc as plsc

N_NODES = 10000
N_EDGES = 320000
D_FEAT = 128
D_OUT = 64
D_ACC = 64

NC = 2
NS = 16
L = 16
NW = NC * NS
E_PER_W = N_EDGES // NW
CHUNK = 80
N_CHUNKS = E_PER_W // CHUNK
NBUF = 5
N_GROUPS = N_CHUNKS // NBUF
N_PAD = 10240
RPS = N_PAD // NS
ZROWS = 128
assert CHUNK % L == 0 and E_PER_W % CHUNK == 0 and N_CHUNKS % NBUF == 0

_mesh = plsc.VectorSubcoreMesh(core_axis_name="c", subcore_axis_name="s")


D_PADOUT = 128


@functools.partial(
    pl.kernel,
    out_type=jax.ShapeDtypeStruct((NC, N_PAD, D_PADOUT), jnp.float32),
    mesh=_mesh,
    compiler_params=pltpu.CompilerParams(use_tc_tiling_on_sc=False,
                                         needs_layout_passes=False),
    scratch_types=[
        pltpu.VMEM((E_PER_W,), jnp.int32),
        [pltpu.VMEM((CHUNK,), jnp.int32) for _ in range(NBUF)],
        [pltpu.VMEM((CHUNK, D_ACC), jnp.float32) for _ in range(NBUF)],
        pltpu.VMEM((ZROWS, D_ACC), jnp.float32),
        pltpu.VMEM((N_PAD,), jnp.float32),
        pltpu.VMEM((NS, RPS), jnp.float32),
        pltpu.VMEM((RPS, L), jnp.float32),
        pltpu.VMEM_SHARED((N_PAD, D_ACC), jnp.float32),
        pltpu.VMEM_SHARED((NS, N_PAD), jnp.float32),
        [pltpu.SemaphoreType.DMA for _ in range(NBUF)],
        [pltpu.SemaphoreType.DMA for _ in range(NBUF)],
        [pltpu.SemaphoreType.DMA for _ in range(NBUF)],
    ],
)
def _sc_segment_sum(y2_hbm, idx_hbm, sum_hbm,
                    srcv, dv, rows, zbuf, degv, dred, dout,
                    acc_sh, deg_sh, gsem, ssem, isem):
    cid = lax.axis_index("c")
    sid = lax.axis_index("s")
    wid = sid * NC + cid

    def wait_rows(sem, buf):
        pltpu.make_async_copy(y2_hbm.at[pl.ds(0, CHUNK)], buf, sem).wait()

    def wait_idx(sem, buf):
        pltpu.make_async_copy(idx_hbm.at[pl.ds(0, CHUNK)], buf, sem).wait()

    @pl.loop(0, ZROWS)
    def _(i):
        @pl.loop(0, D_ACC, step=L)
        def _(j):
            zbuf.at[i, pl.ds(j, L)][...] = jnp.zeros((L,), jnp.float32)

    sbase = wid * E_PER_W
    dbase = N_EDGES + wid * E_PER_W
    for k in range(RPS // ZROWS):
        pltpu.async_copy(zbuf, acc_sh.at[pl.ds(sid * RPS + k * ZROWS, ZROWS)],
                         ssem[k % NBUF])
    pltpu.async_copy(idx_hbm.at[pl.ds(sbase, E_PER_W)], srcv, isem[0])

    @pl.loop(0, N_PAD, step=L)
    def _(i):
        degv.at[pl.ds(i, L)][...] = jnp.zeros((L,), jnp.float32)

    for k in range(RPS // ZROWS):
        pltpu.make_async_copy(y2_hbm.at[pl.ds(0, ZROWS)], zbuf,
                              ssem[k % NBUF]).wait()
    pltpu.make_async_copy(idx_hbm.at[pl.ds(0, E_PER_W)], srcv, isem[0]).wait()

    plsc.subcore_barrier()

    for b in range(NBUF):
        pltpu.async_copy(y2_hbm.at[srcv.at[pl.ds(b * CHUNK, CHUNK)]],
                         rows[b], gsem[b])
        pltpu.async_copy(idx_hbm.at[pl.ds(dbase + b * CHUNK, CHUNK)],
                         dv[b], isem[b])

    @pl.loop(0, N_GROUPS)
    def _(g):
        t0 = g * NBUF
        ones16 = jnp.ones((L,), jnp.float32)
        for b in range(NBUF):
            wait_rows(gsem[b], rows[b])
            wait_idx(isem[b], dv[b])
            pltpu.async_copy(rows[b], acc_sh.at[dv[b]], ssem[b], add=True)
            for k in range(CHUNK // L):
                plsc.addupdate_scatter(degv, [dv[b][pl.ds(k * L, L)]], ones16)

        @pl.when(g < N_GROUPS - 1)
        def _():
            for b in range(NBUF):
                wait_rows(ssem[b], rows[b])
                off = (t0 + NBUF + b) * CHUNK
                pltpu.async_copy(y2_hbm.at[srcv.at[pl.ds(off, CHUNK)]],
                                 rows[b], gsem[b])
                pltpu.async_copy(idx_hbm.at[pl.ds(dbase + off, CHUNK)],
                                 dv[b], isem[b])

    for b in range(NBUF):
        wait_rows(ssem[b], rows[b])

    pltpu.sync_copy(degv, deg_sh.at[sid])

    plsc.subcore_barrier()

    racc = pltpu.async_copy(
        acc_sh.at[pl.ds(sid * RPS, RPS)],
        sum_hbm.at[cid, pl.ds(sid * RPS, RPS), pl.ds(0, D_ACC)], gsem[0])

    pltpu.sync_copy(deg_sh.at[:, pl.ds(sid * RPS, RPS)], dred)
    iota16 = lax.iota(jnp.int32, L)
    zeros16i = jnp.zeros((L,), jnp.int32)

    @pl.loop(0, RPS, step=L)
    def _(j):
        acc = dred[0, pl.ds(j, L)]
        for rr in range(1, NS):
            acc = acc + dred[rr, pl.ds(j, L)]
        plsc.store_scatter(dout, [j + iota16, zeros16i], acc)

    pltpu.sync_copy(dout,
                    sum_hbm.at[cid, pl.ds(sid * RPS, RPS), pl.ds(D_OUT, L)])
    racc.wait()


_PRE_BLK = 2000
_POST_BLK = 2000


def _mm_body(wl_ref, x_ref, y2_ref):
    y2_ref[...] = lax.dot_general(x_ref[...], wl_ref[...],
                                  (((1,), (1,)), ((), ())),
                                  precision=lax.Precision.HIGHEST,
                                  preferred_element_type=jnp.float32)


def _mmb_body(wr_ref, b_ref, x_ref, r_ref):
    r_ref[...] = lax.dot_general(x_ref[...], wr_ref[...],
                                 (((1,), (1,)), ((), ())),
                                 precision=lax.Precision.HIGHEST,
                                 preferred_element_type=jnp.float32) + b_ref[...]


def _post_body(s_ref, r_ref, o_ref):
    s = s_ref[0, :, :D_OUT] + s_ref[1, :, :D_OUT]
    deg = jnp.maximum(s_ref[0, :, D_OUT:D_OUT + 1] + s_ref[1, :, D_OUT:D_OUT + 1],
                      1.0)
    o = s / deg + r_ref[...]
    m = jnp.max(o, axis=1, keepdims=True)
    e = o - m
    lse = jnp.log(jnp.sum(jnp.exp(e), axis=1, keepdims=True))
    o_ref[...] = e - lse


def kernel(x, index, W_l, W_r, b):
    idx_flat = index.astype(jnp.int32).reshape(-1)
    b2 = b.reshape(1, D_OUT).astype(jnp.float32)

    y2 = pl.pallas_call(
        _mm_body,
        grid=(N_NODES // _PRE_BLK,),
        in_specs=[
            pl.BlockSpec((D_OUT, D_FEAT), lambda i: (0, 0)),
            pl.BlockSpec((_PRE_BLK, D_FEAT), lambda i: (i, 0)),
        ],
        out_specs=pl.BlockSpec((_PRE_BLK, D_ACC), lambda i: (i, 0)),
        out_shape=jax.ShapeDtypeStruct((N_NODES, D_ACC), jnp.float32),
    )(W_l, x)

    sums = _sc_segment_sum(y2, idx_flat)

    r = pl.pallas_call(
        _mmb_body,
        grid=(N_NODES // _PRE_BLK,),
        in_specs=[
            pl.BlockSpec((D_OUT, D_FEAT), lambda i: (0, 0)),
            pl.BlockSpec((1, D_OUT), lambda i: (0, 0)),
            pl.BlockSpec((_PRE_BLK, D_FEAT), lambda i: (i, 0)),
        ],
        out_specs=pl.BlockSpec((_PRE_BLK, D_OUT), lambda i: (i, 0)),
        out_shape=jax.ShapeDtypeStruct((N_NODES, D_OUT), jnp.float32),
    )(W_r, b2, x)

    out = pl.pallas_call(
        _post_body,
        grid=(N_NODES // _POST_BLK,),
        in_specs=[
            pl.BlockSpec((NC, _POST_BLK, D_PADOUT), lambda i: (0, i, 0)),
            pl.BlockSpec((_POST_BLK, D_OUT), lambda i: (i, 0)),
        ],
        out_specs=pl.BlockSpec((_POST_BLK, D_OUT), lambda i: (i, 0)),
        out_shape=jax.ShapeDtypeStruct((N_NODES, D_OUT), jnp.float32),
        input_output_aliases={1: 0},
    )(sums, r)

    return out
